# baseline (speedup 1.0000x reference)
import functools

import jax
import jax.numpy as jnp
from jax import lax
from jax.experimental import pallas as pl
from jax.experimental.pallas import tpu as pltpu
from jax.experimental.pallas import tpu_sc as plsc

N = 10000
NPAD = 10240
E = 320000
D, H1, H2 = 128, 32, 64
NC, NS = 2, 16
NW = NC * NS
CHUNK = 128
NCH = E // CHUNK
CPW = NCH // NW
NXTRA = NCH - CPW * NW
RPT = NPAD // NS

_mesh = plsc.VectorSubcoreMesh(core_axis_name="c", subcore_axis_name="s")
_sc_params = pltpu.CompilerParams(use_tc_tiling_on_sc=False)


@functools.partial(
    pl.kernel,
    out_type=jax.ShapeDtypeStruct((NC, NPAD), jnp.float32),
    mesh=_mesh,
    scratch_types=[
        pltpu.VMEM((CPW + 1, CHUNK), jnp.int32),
        pltpu.VMEM((CHUNK,), jnp.float32),
        pltpu.VMEM((RPT,), jnp.float32),
        pltpu.VMEM_SHARED((NPAD,), jnp.float32),
        pltpu.SemaphoreType.DMA,
    ],
    compiler_params=_sc_params,
)
def _deg_kernel(ei_hbm, out_hbm, dst_v, ones_v, slab_v, acc, dsem):
    cid = lax.axis_index("c")
    tid = lax.axis_index("s")
    wid = tid * NC + cid
    for k in range(CHUNK // 16):
        ones_v[pl.ds(16 * k, 16)] = jnp.ones((16,), jnp.float32)
    for k in range(RPT // 16):
        slab_v[pl.ds(16 * k, 16)] = jnp.zeros((16,), jnp.float32)
    pltpu.sync_copy(slab_v, acc.at[pl.ds(tid * RPT, RPT)])
    pltpu.sync_copy(ei_hbm.at[1, pl.ds(wid * CPW, CPW)],
                    dst_v.at[pl.ds(0, CPW)])

    @pl.when(wid < NXTRA)
    def _():
        pltpu.sync_copy(ei_hbm.at[1, NW * CPW + wid], dst_v.at[CPW])

    nch = jnp.where(wid < NXTRA, CPW + 1, CPW)
    plsc.subcore_barrier()

    def body(j, carry):
        pltpu.async_copy(ones_v, acc.at[dst_v.at[j]], dsem, add=True)
        return carry

    lax.fori_loop(0, nch, body, 0)

    def drain(j, carry):
        pltpu.make_async_copy(ones_v, acc.at[dst_v.at[j]], dsem).wait()
        return carry

    lax.fori_loop(0, nch, drain, 0)
    plsc.subcore_barrier()
    pltpu.sync_copy(acc.at[pl.ds(tid * RPT, RPT)], slab_v)
    pltpu.sync_copy(slab_v, out_hbm.at[cid, pl.ds(tid * RPT, RPT)])


def _make_scatter(F, nbuf, hq0=0, hq1=0):
    @functools.partial(
        pl.kernel,
        out_type=jax.ShapeDtypeStruct((NC, NPAD, F), jnp.float32),
        mesh=_mesh,
        scratch_types=[
            pltpu.VMEM((CPW + 1, CHUNK), jnp.int32),
            pltpu.VMEM((CPW + 1, CHUNK), jnp.int32),
            [pltpu.VMEM((CHUNK, F), jnp.float32) for _ in range(nbuf)],
            pltpu.VMEM_SHARED((NPAD, F), jnp.float32),
            pltpu.VMEM_SHARED((NPAD, F), jnp.float32),
            [pltpu.SemaphoreType.DMA for _ in range(nbuf)],
            [pltpu.SemaphoreType.DMA for _ in range(nbuf)],
        ],
        compiler_params=_sc_params,
    )
    def _k(y_hbm, ei_hbm, out_hbm, src_v, dst_v, rows, acc, y_sp, gsem, ssem):
        cid = lax.axis_index("c")
        tid = lax.axis_index("s")
        wid = tid * NC + cid
        base = tid * RPT
        pltpu.sync_copy(y_hbm.at[pl.ds(base, RPT)], y_sp.at[pl.ds(base, RPT)])
        pltpu.sync_copy(y_hbm.at[pl.ds(base, RPT)], acc.at[pl.ds(base, RPT)])
        pltpu.sync_copy(ei_hbm.at[0, pl.ds(wid * CPW, CPW)],
                        src_v.at[pl.ds(0, CPW)])
        pltpu.sync_copy(ei_hbm.at[1, pl.ds(wid * CPW, CPW)],
                        dst_v.at[pl.ds(0, CPW)])

        @pl.when(wid < NXTRA)
        def _():
            pltpu.sync_copy(ei_hbm.at[0, NW * CPW + wid], src_v.at[CPW])
            pltpu.sync_copy(ei_hbm.at[1, NW * CPW + wid], dst_v.at[CPW])

        plsc.subcore_barrier()

        if hq0 or hq1:
            hq = jnp.where(cid == 0, hq0, hq1)

            def gather(j, b):
                use_hbm = lax.rem(jnp.int32(j), 4) < hq

                @pl.when(use_hbm)
                def _():
                    pltpu.async_copy(y_hbm.at[src_v.at[j]], rows[b], gsem[b])

                @pl.when(jnp.logical_not(use_hbm))
                def _():
                    pltpu.async_copy(y_sp.at[src_v.at[j]], rows[b], gsem[b])
        else:
            def gather(j, b):
                pltpu.async_copy(y_sp.at[src_v.at[j]], rows[b], gsem[b])

        def gather_wait(j, b):
            pltpu.make_async_copy(y_sp.at[src_v.at[j]], rows[b],
                                  gsem[b]).wait()

        def scat(j, b):
            pltpu.async_copy(rows[b], acc.at[dst_v.at[j]], ssem[b], add=True)

        def scat_wait(j, b):
            pltpu.make_async_copy(rows[b], acc.at[dst_v.at[j]],
                                  ssem[b]).wait()

        for b in range(nbuf):
            gather(b, b)

        def body(i, carry):
            j = nbuf * i
            for b in range(nbuf):
                gather_wait(j + b, b)
                scat(j + b, b)
            for b in range(nbuf):
                scat_wait(j + b, b)
                gather(j + nbuf + b, b)
            return carry

        lax.fori_loop(0, CPW // nbuf - 1, body, 0)
        last = CPW - nbuf
        for b in range(nbuf):
            gather_wait(last + b, b)
            scat(last + b, b)
        for b in range(nbuf):
            scat_wait(last + b, b)

        @pl.when(wid < NXTRA)
        def _():
            gather(CPW, 0)
            gather_wait(CPW, 0)
            scat(CPW, 0)
            scat_wait(CPW, 0)

        plsc.subcore_barrier()
        pltpu.sync_copy(acc.at[pl.ds(base, RPT)],
                        out_hbm.at[cid, pl.ds(base, RPT)])

    return _k


_scatter32 = _make_scatter(H1, 6)
_scatter64 = _make_scatter(H2, 2, 2, 1)


def _tc_xw(x_ref, w1_ref, xw_ref):
    xw_ref[...] = jnp.dot(x_ref[...], w1_ref[...],
                          preferred_element_type=jnp.float32)


def _tc_first(xw_ref, degs_ref, y1_ref, dinv_ref):
    deg = degs_ref[:, 0:1] + degs_ref[:, 1:2] + 1.0
    dinv = lax.rsqrt(deg)
    dinv_ref[...] = dinv
    y1_ref[pl.ds(0, N)] = xw_ref[...] * dinv[:N]
    y1_ref[pl.ds(N, NPAD - N)] = jnp.zeros((NPAD - N, H1), jnp.float32)


def _tc_mid(accs_ref, y1_ref, dinv_ref, b1_ref, w2_ref, y2_ref):
    s = accs_ref[0] + accs_ref[1] - y1_ref[...]
    z = jnp.maximum(s * dinv_ref[...] + b1_ref[...], 0.0)
    y2_ref[...] = (
        jnp.dot(z, w2_ref[...], preferred_element_type=jnp.float32)
        * dinv_ref[...]
    )


def _tc_last(accs_ref, y2_ref, dinv_ref, b2_ref, wo1_ref, bo1_ref,
             wo2_ref, bo2_ref, out_ref):
    s = accs_ref[0] + accs_ref[1] - y2_ref[...]
    z2 = jnp.maximum(s * dinv_ref[...] + b2_ref[...], 0.0)
    t = jnp.maximum(
        jnp.dot(z2, wo1_ref[...], preferred_element_type=jnp.float32)
        + bo1_ref[...], 0.0)
    h = (jnp.dot(t, wo2_ref[...], preferred_element_type=jnp.float32)
         + bo2_ref[...])
    rid = lax.broadcasted_iota(jnp.int32, (NPAD, 1), 0)
    h = jnp.where(rid < N, h, 0.0)
    out_ref[...] = jnp.sum(h).reshape(1, 1) / N


def kernel(x, ei, num_nodes, W1, b1, W2, b2, Wo1, bo1, Wo2, bo2):
    ei_r = ei.reshape(2, NCH, CHUNK)

    degs = _deg_kernel(ei_r)
    xw = pl.pallas_call(
        _tc_xw,
        out_shape=jax.ShapeDtypeStruct((N, H1), jnp.float32),
    )(x, W1)
    degs_t = degs.T

    y1, dinv = pl.pallas_call(
        _tc_first,
        out_shape=[
            jax.ShapeDtypeStruct((NPAD, H1), jnp.float32),
            jax.ShapeDtypeStruct((NPAD, 1), jnp.float32),
        ],
    )(xw, degs_t)

    accs1 = _scatter32(y1, ei_r)

    y2 = pl.pallas_call(
        _tc_mid,
        out_shape=jax.ShapeDtypeStruct((NPAD, H2), jnp.float32),
    )(accs1, y1, dinv, b1, W2)

    accs2 = _scatter64(y2, ei_r)

    out = pl.pallas_call(
        _tc_last,
        out_shape=jax.ShapeDtypeStruct((1, 1), jnp.float32),
    )(accs2, y2, dinv, b2, Wo1, bo1, Wo2, bo2)

    return out.reshape(1)


# speedup vs baseline: 1.1308x; 1.1308x over previous
import functools

import jax
import jax.numpy as jnp
from jax import lax
from jax.experimental import pallas as pl
from jax.experimental.pallas import tpu as pltpu
from jax.experimental.pallas import tpu_sc as plsc

N = 10000
NPAD = 10240
E = 320000
D, H1, H2 = 128, 32, 64
NC, NS = 2, 16
NW = NC * NS
CHUNK = 128
NCH = E // CHUNK
CPW = NCH // NW
NXTRA = NCH - CPW * NW
RPT = NPAD // NS

_mesh = plsc.VectorSubcoreMesh(core_axis_name="c", subcore_axis_name="s")
_sc_params = pltpu.CompilerParams(use_tc_tiling_on_sc=False)


@functools.partial(
    pl.kernel,
    out_type=jax.ShapeDtypeStruct((NC, NPAD), jnp.float32),
    mesh=_mesh,
    scratch_types=[
        pltpu.VMEM((CPW + 1, CHUNK), jnp.int32),
        pltpu.VMEM((CHUNK,), jnp.float32),
        pltpu.VMEM((RPT,), jnp.float32),
        pltpu.VMEM_SHARED((NPAD,), jnp.float32),
        pltpu.SemaphoreType.DMA,
    ],
    compiler_params=_sc_params,
)
def _deg_kernel(ei_hbm, out_hbm, dst_v, ones_v, slab_v, acc, dsem):
    cid = lax.axis_index("c")
    tid = lax.axis_index("s")
    wid = tid * NC + cid
    for k in range(CHUNK // 16):
        ones_v[pl.ds(16 * k, 16)] = jnp.ones((16,), jnp.float32)
    for k in range(RPT // 16):
        slab_v[pl.ds(16 * k, 16)] = jnp.zeros((16,), jnp.float32)
    pltpu.sync_copy(slab_v, acc.at[pl.ds(tid * RPT, RPT)])
    pltpu.sync_copy(ei_hbm.at[1, pl.ds(wid * CPW, CPW)],
                    dst_v.at[pl.ds(0, CPW)])

    @pl.when(wid < NXTRA)
    def _():
        pltpu.sync_copy(ei_hbm.at[1, NW * CPW + wid], dst_v.at[CPW])

    nch = jnp.where(wid < NXTRA, CPW + 1, CPW)
    plsc.subcore_barrier()

    def body(j, carry):
        pltpu.async_copy(ones_v, acc.at[dst_v.at[j]], dsem, add=True)
        return carry

    lax.fori_loop(0, nch, body, 0)

    def drain(j, carry):
        pltpu.make_async_copy(ones_v, acc.at[dst_v.at[j]], dsem).wait()
        return carry

    lax.fori_loop(0, nch, drain, 0)
    plsc.subcore_barrier()
    pltpu.sync_copy(acc.at[pl.ds(tid * RPT, RPT)], slab_v)
    pltpu.sync_copy(slab_v, out_hbm.at[cid, pl.ds(tid * RPT, RPT)])


def _make_scatter(F, nbuf, hq0=0, hq1=0):
    @functools.partial(
        pl.kernel,
        out_type=jax.ShapeDtypeStruct((NC, NPAD, F), jnp.float32),
        mesh=_mesh,
        scratch_types=[
            pltpu.VMEM((CPW + 1, CHUNK), jnp.int32),
            pltpu.VMEM((CPW + 1, CHUNK), jnp.int32),
            [pltpu.VMEM((CHUNK, F), jnp.float32) for _ in range(nbuf)],
            pltpu.VMEM_SHARED((NPAD, F), jnp.float32),
            pltpu.VMEM_SHARED((NPAD, F), jnp.float32),
            [pltpu.SemaphoreType.DMA for _ in range(nbuf)],
            [pltpu.SemaphoreType.DMA for _ in range(nbuf)],
        ],
        compiler_params=_sc_params,
    )
    def _k(y_hbm, ei_hbm, out_hbm, src_v, dst_v, rows, acc, y_sp, gsem, ssem):
        cid = lax.axis_index("c")
        tid = lax.axis_index("s")
        wid = tid * NC + cid
        base = tid * RPT
        pltpu.sync_copy(y_hbm.at[pl.ds(base, RPT)], y_sp.at[pl.ds(base, RPT)])
        pltpu.sync_copy(y_hbm.at[pl.ds(base, RPT)], acc.at[pl.ds(base, RPT)])
        pltpu.sync_copy(ei_hbm.at[0, pl.ds(wid * CPW, CPW)],
                        src_v.at[pl.ds(0, CPW)])
        pltpu.sync_copy(ei_hbm.at[1, pl.ds(wid * CPW, CPW)],
                        dst_v.at[pl.ds(0, CPW)])

        @pl.when(wid < NXTRA)
        def _():
            pltpu.sync_copy(ei_hbm.at[0, NW * CPW + wid], src_v.at[CPW])
            pltpu.sync_copy(ei_hbm.at[1, NW * CPW + wid], dst_v.at[CPW])

        plsc.subcore_barrier()

        if hq0 or hq1:
            hq = jnp.where(cid == 0, hq0, hq1)

            def gather(j, b):
                use_hbm = lax.rem(jnp.int32(j), 4) < hq

                @pl.when(use_hbm)
                def _():
                    pltpu.async_copy(y_hbm.at[src_v.at[j]], rows[b], gsem[b])

                @pl.when(jnp.logical_not(use_hbm))
                def _():
                    pltpu.async_copy(y_sp.at[src_v.at[j]], rows[b], gsem[b])
        else:
            def gather(j, b):
                pltpu.async_copy(y_sp.at[src_v.at[j]], rows[b], gsem[b])

        def gather_wait(j, b):
            pltpu.make_async_copy(y_sp.at[src_v.at[j]], rows[b],
                                  gsem[b]).wait()

        def scat(j, b):
            pltpu.async_copy(rows[b], acc.at[dst_v.at[j]], ssem[b], add=True)

        def scat_wait(j, b):
            pltpu.make_async_copy(rows[b], acc.at[dst_v.at[j]],
                                  ssem[b]).wait()

        for b in range(nbuf):
            gather(b, b)

        def body(i, carry):
            j = nbuf * i
            for b in range(nbuf):
                gather_wait(j + b, b)
                scat(j + b, b)
            for b in range(nbuf):
                scat_wait(j + b, b)
                gather(j + nbuf + b, b)
            return carry

        lax.fori_loop(0, CPW // nbuf - 1, body, 0)
        last = CPW - nbuf
        for b in range(nbuf):
            gather_wait(last + b, b)
            scat(last + b, b)
        for b in range(nbuf):
            scat_wait(last + b, b)

        @pl.when(wid < NXTRA)
        def _():
            gather(CPW, 0)
            gather_wait(CPW, 0)
            scat(CPW, 0)
            scat_wait(CPW, 0)

        plsc.subcore_barrier()
        pltpu.sync_copy(acc.at[pl.ds(base, RPT)],
                        out_hbm.at[cid, pl.ds(base, RPT)])

    return _k


_scatter32 = _make_scatter(H1, 13)
_scatter64 = _make_scatter(H2, 2)


def _tc_xw(x_ref, w1_ref, xw_ref):
    xw_ref[...] = jnp.dot(x_ref[...], w1_ref[...],
                          preferred_element_type=jnp.float32)


def _tc_first(xw_ref, degs_ref, y1_ref, dinv_ref):
    deg = degs_ref[:, 0:1] + degs_ref[:, 1:2] + 1.0
    dinv = lax.rsqrt(deg)
    dinv_ref[...] = dinv
    y1_ref[pl.ds(0, N)] = xw_ref[...] * dinv[:N]
    y1_ref[pl.ds(N, NPAD - N)] = jnp.zeros((NPAD - N, H1), jnp.float32)


def _tc_mid(accs_ref, y1_ref, dinv_ref, b1_ref, w2_ref, y2_ref):
    s = accs_ref[0] + accs_ref[1] - y1_ref[...]
    z = jnp.maximum(s * dinv_ref[...] + b1_ref[...], 0.0)
    y2_ref[...] = (
        jnp.dot(z, w2_ref[...], preferred_element_type=jnp.float32)
        * dinv_ref[...]
    )


def _tc_last(accs_ref, y2_ref, dinv_ref, b2_ref, wo1_ref, bo1_ref,
             wo2_ref, bo2_ref, out_ref):
    s = accs_ref[0] + accs_ref[1] - y2_ref[...]
    z2 = jnp.maximum(s * dinv_ref[...] + b2_ref[...], 0.0)
    t = jnp.maximum(
        jnp.dot(z2, wo1_ref[...], preferred_element_type=jnp.float32)
        + bo1_ref[...], 0.0)
    h = (jnp.dot(t, wo2_ref[...], preferred_element_type=jnp.float32)
         + bo2_ref[...])
    rid = lax.broadcasted_iota(jnp.int32, (NPAD, 1), 0)
    h = jnp.where(rid < N, h, 0.0)
    out_ref[...] = jnp.sum(h).reshape(1, 1) / N


def kernel(x, ei, num_nodes, W1, b1, W2, b2, Wo1, bo1, Wo2, bo2):
    ei_r = ei.reshape(2, NCH, CHUNK)

    degs = _deg_kernel(ei_r)
    xw = pl.pallas_call(
        _tc_xw,
        out_shape=jax.ShapeDtypeStruct((N, H1), jnp.float32),
    )(x, W1)
    degs_t = degs.T

    y1, dinv = pl.pallas_call(
        _tc_first,
        out_shape=[
            jax.ShapeDtypeStruct((NPAD, H1), jnp.float32),
            jax.ShapeDtypeStruct((NPAD, 1), jnp.float32),
        ],
    )(xw, degs_t)

    accs1 = _scatter32(y1, ei_r)

    y2 = pl.pallas_call(
        _tc_mid,
        out_shape=jax.ShapeDtypeStruct((NPAD, H2), jnp.float32),
    )(accs1, y1, dinv, b1, W2)

    accs2 = _scatter64(y2, ei_r)

    out = pl.pallas_call(
        _tc_last,
        out_shape=jax.ShapeDtypeStruct((1, 1), jnp.float32),
    )(accs2, y2, dinv, b2, Wo1, bo1, Wo2, bo2)

    return out.reshape(1)


# speedup vs baseline: 1.1518x; 1.0186x over previous
import functools

import jax
import jax.numpy as jnp
from jax import lax
from jax.experimental import pallas as pl
from jax.experimental.pallas import tpu as pltpu
from jax.experimental.pallas import tpu_sc as plsc

N = 10000
NPAD = 10240
E = 320000
D, H1, H2 = 128, 32, 64
NC, NS = 2, 16
NW = NC * NS
CHUNK = 128
NCH = E // CHUNK
CPW = NCH // NW
NXTRA = NCH - CPW * NW
RPT = NPAD // NS

_mesh = plsc.VectorSubcoreMesh(core_axis_name="c", subcore_axis_name="s")
_sc_params = pltpu.CompilerParams(use_tc_tiling_on_sc=False)


@functools.partial(
    pl.kernel,
    out_type=jax.ShapeDtypeStruct((NC, NPAD), jnp.float32),
    mesh=_mesh,
    scratch_types=[
        pltpu.VMEM((CPW + 1, CHUNK), jnp.int32),
        pltpu.VMEM((CHUNK,), jnp.float32),
        pltpu.VMEM((RPT,), jnp.float32),
        pltpu.VMEM_SHARED((NPAD,), jnp.float32),
        pltpu.SemaphoreType.DMA,
    ],
    compiler_params=_sc_params,
)
def _deg_kernel(ei_hbm, out_hbm, dst_v, ones_v, slab_v, acc, dsem):
    cid = lax.axis_index("c")
    tid = lax.axis_index("s")
    wid = tid * NC + cid
    for k in range(CHUNK // 16):
        ones_v[pl.ds(16 * k, 16)] = jnp.ones((16,), jnp.float32)
    for k in range(RPT // 16):
        slab_v[pl.ds(16 * k, 16)] = jnp.zeros((16,), jnp.float32)
    pltpu.sync_copy(slab_v, acc.at[pl.ds(tid * RPT, RPT)])
    pltpu.sync_copy(ei_hbm.at[1, pl.ds(wid * CPW, CPW)],
                    dst_v.at[pl.ds(0, CPW)])

    @pl.when(wid < NXTRA)
    def _():
        pltpu.sync_copy(ei_hbm.at[1, NW * CPW + wid], dst_v.at[CPW])

    nch = jnp.where(wid < NXTRA, CPW + 1, CPW)
    plsc.subcore_barrier()

    def body(j, carry):
        pltpu.async_copy(ones_v, acc.at[dst_v.at[j]], dsem, add=True)
        return carry

    lax.fori_loop(0, nch, body, 0)

    def drain(j, carry):
        pltpu.make_async_copy(ones_v, acc.at[dst_v.at[j]], dsem).wait()
        return carry

    lax.fori_loop(0, nch, drain, 0)
    plsc.subcore_barrier()
    pltpu.sync_copy(acc.at[pl.ds(tid * RPT, RPT)], slab_v)
    pltpu.sync_copy(slab_v, out_hbm.at[cid, pl.ds(tid * RPT, RPT)])


def _make_scatter(F, nbuf, hq0=0, hq1=0):
    @functools.partial(
        pl.kernel,
        out_type=jax.ShapeDtypeStruct((NC, NPAD, F), jnp.float32),
        mesh=_mesh,
        scratch_types=[
            pltpu.VMEM((CPW + 1, CHUNK), jnp.int32),
            pltpu.VMEM((CPW + 1, CHUNK), jnp.int32),
            [pltpu.VMEM((CHUNK, F), jnp.float32) for _ in range(nbuf)],
            pltpu.VMEM_SHARED((NPAD, F), jnp.float32),
            pltpu.VMEM_SHARED((NPAD, F), jnp.float32),
            [pltpu.SemaphoreType.DMA for _ in range(nbuf)],
            [pltpu.SemaphoreType.DMA for _ in range(nbuf)],
        ],
        compiler_params=_sc_params,
    )
    def _k(y_hbm, ei_hbm, out_hbm, src_v, dst_v, rows, acc, y_sp, gsem, ssem):
        cid = lax.axis_index("c")
        tid = lax.axis_index("s")
        wid = tid * NC + cid
        base = tid * RPT
        c_ysp = pltpu.async_copy(y_hbm.at[pl.ds(base, RPT)],
                                 y_sp.at[pl.ds(base, RPT)], gsem[0])
        c_acc = pltpu.async_copy(y_hbm.at[pl.ds(base, RPT)],
                                 acc.at[pl.ds(base, RPT)], gsem[1])
        c_src = pltpu.async_copy(ei_hbm.at[0, pl.ds(wid * CPW, CPW)],
                                 src_v.at[pl.ds(0, CPW)], ssem[0])
        c_dst = pltpu.async_copy(ei_hbm.at[1, pl.ds(wid * CPW, CPW)],
                                 dst_v.at[pl.ds(0, CPW)], ssem[1])
        c_ysp.wait()
        c_acc.wait()
        c_src.wait()
        c_dst.wait()

        @pl.when(wid < NXTRA)
        def _():
            pltpu.sync_copy(ei_hbm.at[0, NW * CPW + wid], src_v.at[CPW])
            pltpu.sync_copy(ei_hbm.at[1, NW * CPW + wid], dst_v.at[CPW])

        plsc.subcore_barrier()

        if hq0 or hq1:
            hq = jnp.where(cid == 0, hq0, hq1)

            def gather(j, b):
                use_hbm = lax.rem(jnp.int32(j), 4) < hq

                @pl.when(use_hbm)
                def _():
                    pltpu.async_copy(y_hbm.at[src_v.at[j]], rows[b], gsem[b])

                @pl.when(jnp.logical_not(use_hbm))
                def _():
                    pltpu.async_copy(y_sp.at[src_v.at[j]], rows[b], gsem[b])
        else:
            def gather(j, b):
                pltpu.async_copy(y_sp.at[src_v.at[j]], rows[b], gsem[b])

        def gather_wait(j, b):
            pltpu.make_async_copy(y_sp.at[src_v.at[j]], rows[b],
                                  gsem[b]).wait()

        def scat(j, b):
            pltpu.async_copy(rows[b], acc.at[dst_v.at[j]], ssem[b], add=True)

        def scat_wait(j, b):
            pltpu.make_async_copy(rows[b], acc.at[dst_v.at[j]],
                                  ssem[b]).wait()

        for b in range(nbuf):
            gather(b, b)

        def body(i, carry):
            j = nbuf * i
            for b in range(nbuf):
                gather_wait(j + b, b)
                scat(j + b, b)
            for b in range(nbuf):
                scat_wait(j + b, b)
                gather(j + nbuf + b, b)
            return carry

        lax.fori_loop(0, CPW // nbuf - 1, body, 0)
        last = CPW - nbuf
        for b in range(nbuf):
            gather_wait(last + b, b)
            scat(last + b, b)
        for b in range(nbuf):
            scat_wait(last + b, b)

        @pl.when(wid < NXTRA)
        def _():
            gather(CPW, 0)
            gather_wait(CPW, 0)
            scat(CPW, 0)
            scat_wait(CPW, 0)

        plsc.subcore_barrier()
        pltpu.sync_copy(acc.at[pl.ds(base, RPT)],
                        out_hbm.at[cid, pl.ds(base, RPT)])

    return _k


_scatter32 = _make_scatter(H1, 13)
_scatter64 = _make_scatter(H2, 2)


def _tc_xw(x_ref, w1_ref, xw_ref):
    xw_ref[...] = jnp.dot(x_ref[...], w1_ref[...],
                          preferred_element_type=jnp.float32)


def _tc_first(xw_ref, degs_ref, y1_ref, dinv_ref):
    deg = degs_ref[:, 0:1] + degs_ref[:, 1:2] + 1.0
    dinv = lax.rsqrt(deg)
    dinv_ref[...] = dinv
    y1_ref[pl.ds(0, N)] = xw_ref[...] * dinv[:N]
    y1_ref[pl.ds(N, NPAD - N)] = jnp.zeros((NPAD - N, H1), jnp.float32)


def _tc_mid(accs_ref, y1_ref, dinv_ref, b1_ref, w2_ref, y2_ref):
    s = accs_ref[0] + accs_ref[1] - y1_ref[...]
    z = jnp.maximum(s * dinv_ref[...] + b1_ref[...], 0.0)
    y2_ref[...] = (
        jnp.dot(z, w2_ref[...], preferred_element_type=jnp.float32)
        * dinv_ref[...]
    )


def _tc_last(accs_ref, y2_ref, dinv_ref, b2_ref, wo1_ref, bo1_ref,
             wo2_ref, bo2_ref, out_ref):
    s = accs_ref[0] + accs_ref[1] - y2_ref[...]
    z2 = jnp.maximum(s * dinv_ref[...] + b2_ref[...], 0.0)
    t = jnp.maximum(
        jnp.dot(z2, wo1_ref[...], preferred_element_type=jnp.float32)
        + bo1_ref[...], 0.0)
    h = (jnp.dot(t, wo2_ref[...], preferred_element_type=jnp.float32)
         + bo2_ref[...])
    rid = lax.broadcasted_iota(jnp.int32, (NPAD, 1), 0)
    h = jnp.where(rid < N, h, 0.0)
    out_ref[...] = jnp.sum(h).reshape(1, 1) / N


def kernel(x, ei, num_nodes, W1, b1, W2, b2, Wo1, bo1, Wo2, bo2):
    ei_r = ei.reshape(2, NCH, CHUNK)

    degs = _deg_kernel(ei_r)
    xw = pl.pallas_call(
        _tc_xw,
        out_shape=jax.ShapeDtypeStruct((N, H1), jnp.float32),
    )(x, W1)
    degs_t = degs.T

    y1, dinv = pl.pallas_call(
        _tc_first,
        out_shape=[
            jax.ShapeDtypeStruct((NPAD, H1), jnp.float32),
            jax.ShapeDtypeStruct((NPAD, 1), jnp.float32),
        ],
    )(xw, degs_t)

    accs1 = _scatter32(y1, ei_r)

    y2 = pl.pallas_call(
        _tc_mid,
        out_shape=jax.ShapeDtypeStruct((NPAD, H2), jnp.float32),
    )(accs1, y1, dinv, b1, W2)

    accs2 = _scatter64(y2, ei_r)

    out = pl.pallas_call(
        _tc_last,
        out_shape=jax.ShapeDtypeStruct((1, 1), jnp.float32),
    )(accs2, y2, dinv, b2, Wo1, bo1, Wo2, bo2)

    return out.reshape(1)
